# baseline (speedup 1.0000x reference)
import jax
import jax.numpy as jnp
from jax.experimental import pallas as pl
from jax.experimental.pallas import tpu as pltpu

_K = 5


def _chunk_copy(x_ref, xo_ref):
    xo_ref[...] = x_ref[...]


def _copy_body(x_hbm, b_ref, xo_hbm, bo_ref):
    n, d = x_hbm.shape
    rows = n // _K
    pltpu.emit_pipeline(
        _chunk_copy,
        grid=(_K,),
        in_specs=[pl.BlockSpec((rows, d), lambda i: (i, 0))],
        out_specs=[pl.BlockSpec((rows, d), lambda i: (i, 0))],
    )(x_hbm, xo_hbm)
    bo_ref[...] = b_ref[...]


def kernel(x, edge_index, batch):
    del edge_index
    xo, bo = pl.pallas_call(
        _copy_body,
        in_specs=[
            pl.BlockSpec(memory_space=pltpu.MemorySpace.HBM),
            pl.BlockSpec(memory_space=pltpu.MemorySpace.VMEM),
        ],
        out_specs=(
            pl.BlockSpec(memory_space=pltpu.MemorySpace.HBM),
            pl.BlockSpec(memory_space=pltpu.MemorySpace.VMEM),
        ),
        out_shape=(
            jax.ShapeDtypeStruct(x.shape, x.dtype),
            jax.ShapeDtypeStruct(batch.shape, batch.dtype),
        ),
    )(x, batch)
    return (xo, bo)
